# Initial kernel scaffold; baseline (speedup 1.0000x reference)
import jax
import jax.numpy as jnp
from jax.experimental import pallas as pl


def kernel(node_x, edge_index, edge_attr, params):
    raise NotImplementedError("write your pallas kernel here")


# speedup vs baseline: 2.2863x; 2.2863x over previous
import functools

import jax
import jax.numpy as jnp
from jax import lax
from jax.experimental import pallas as pl
from jax.experimental.pallas import tpu as pltpu
from jax.experimental.pallas import tpu_sc as plsc

N = 10000
E = 320000
D = 128
ED = 16
N_LAYERS = 3
EPS = 1e-5

NB = 1000
EB = 4000

NC = 2
NS = 16
NW = NC * NS
EPW = E // NW
CH = 80
NCH = EPW // CH
NPAD = 10240
RPS = NPAD // NS


def _ln(h, g, b):
    mu = jnp.mean(h, axis=1, keepdims=True)
    var = jnp.mean((h - mu) ** 2, axis=1, keepdims=True)
    return (h - mu) * lax.rsqrt(var + EPS) * g + b


def _node_init_body(x_ref, w1, b1, w2, b2, w3, b3, lng, lnb, wm, bm,
                    h_out, a_out):
    x = x_ref[...]
    h = jnp.maximum(jnp.dot(x, w1[...], preferred_element_type=jnp.float32) + b1[...], 0.0)
    h = jnp.maximum(jnp.dot(h, w2[...], preferred_element_type=jnp.float32) + b2[...], 0.0)
    h = jnp.dot(h, w3[...], preferred_element_type=jnp.float32) + b3[...]
    hn = _ln(h, lng[...], lnb[...])
    h_out[...] = hn
    a_out[...] = jnp.dot(hn, wm[...], preferred_element_type=jnp.float32) + bm[...]


def _edge_body(g_ref, ea_ref, w1e, w2, b2, w3, b3, m_out):
    h1 = jnp.maximum(
        g_ref[...] + jnp.dot(ea_ref[...], w1e[...], preferred_element_type=jnp.float32),
        0.0)
    h2 = jnp.maximum(jnp.dot(h1, w2[...], preferred_element_type=jnp.float32) + b2[...], 0.0)
    m_out[...] = jnp.dot(h2, w3[...], preferred_element_type=jnp.float32) + b3[...]


def _update_body(h_ref, p0_ref, p1_ref, w1h, w1a, b1, w2, b2, w3, b3,
                 lng, lnb, wm, bm, h_out, a_out, gsum_out):
    h = h_ref[...]
    agg = p0_ref[...] + p1_ref[...]
    u = jnp.maximum(
        jnp.dot(h, w1h[...], preferred_element_type=jnp.float32)
        + jnp.dot(agg, w1a[...], preferred_element_type=jnp.float32) + b1[...], 0.0)
    u = jnp.maximum(jnp.dot(u, w2[...], preferred_element_type=jnp.float32) + b2[...], 0.0)
    u = jnp.dot(u, w3[...], preferred_element_type=jnp.float32) + b3[...]
    hn = _ln(h + u, lng[...], lnb[...])
    h_out[...] = hn
    a_out[...] = jnp.dot(hn, wm[...], preferred_element_type=jnp.float32) + bm[...]

    @pl.when(pl.program_id(0) == 0)
    def _():
        gsum_out[...] = jnp.zeros_like(gsum_out)
    gsum_out[...] += jnp.sum(hn, axis=0, keepdims=True)


def _full(shape):
    return pl.BlockSpec(shape, lambda i: (0, 0))


def _node_init_call(x, w1, b1, w2, b2, w3, b3, lng, lnb, wm, bm):
    grid = (N // NB,)
    row = pl.BlockSpec((NB, D), lambda i: (i, 0))
    return pl.pallas_call(
        _node_init_body,
        grid=grid,
        in_specs=[row, _full((D, D)), _full((1, D)), _full((D, D)), _full((1, D)),
                  _full((D, D)), _full((1, D)), _full((1, D)), _full((1, D)),
                  _full((D, D)), _full((1, D))],
        out_specs=[row, row],
        out_shape=[jax.ShapeDtypeStruct((N, D), jnp.float32),
                   jax.ShapeDtypeStruct((N, D), jnp.float32)],
    )(x, w1, b1, w2, b2, w3, b3, lng, lnb, wm, bm)


def _edge_call(g, ea, w1e, w2, b2, w3, b3):
    grid = (E // EB,)
    row = pl.BlockSpec((EB, D), lambda i: (i, 0))
    erow = pl.BlockSpec((EB, ED), lambda i: (i, 0))
    return pl.pallas_call(
        _edge_body,
        grid=grid,
        in_specs=[row, erow, _full((ED, D)), _full((D, D)), _full((1, D)),
                  _full((D, D)), _full((1, D))],
        out_specs=row,
        out_shape=jax.ShapeDtypeStruct((E, D), jnp.float32),
    )(g, ea, w1e, w2, b2, w3, b3)


def _update_call(h, p0, p1, w1h, w1a, b1, w2, b2, w3, b3, lng, lnb, wm, bm):
    grid = (N // NB,)
    row = pl.BlockSpec((NB, D), lambda i: (i, 0))
    return pl.pallas_call(
        _update_body,
        grid=grid,
        in_specs=[row, row, row,
                  _full((D, D)), _full((D, D)), _full((1, D)),
                  _full((D, D)), _full((1, D)), _full((D, D)), _full((1, D)),
                  _full((1, D)), _full((1, D)), _full((D, D)), _full((1, D))],
        out_specs=[row, row, _full((1, D))],
        out_shape=[jax.ShapeDtypeStruct((N, D), jnp.float32),
                   jax.ShapeDtypeStruct((N, D), jnp.float32),
                   jax.ShapeDtypeStruct((1, D), jnp.float32)],
        compiler_params=pltpu.CompilerParams(
            dimension_semantics=("arbitrary",)),
    )(h, p0, p1, w1h, w1a, b1, w2, b2, w3, b3, lng, lnb, wm, bm)


@functools.cache
def _sc_calls():
    mesh = plsc.VectorSubcoreMesh(core_axis_name="c", subcore_axis_name="s")

    @functools.partial(
        pl.kernel,
        mesh=mesh,
        out_type=jax.ShapeDtypeStruct((E, D), jnp.float32),
        scratch_types=[
            pltpu.VMEM((CH,), jnp.int32),
            pltpu.VMEM((CH, D), jnp.float32),
            pltpu.SemaphoreType.DMA,
        ],
    )
    def sc_gather(table_hbm, idx_hbm, out_hbm, idx_v, rows_v, sem):
        wid = lax.axis_index("s") * NC + lax.axis_index("c")

        def body(i, carry):
            base = wid * EPW + i * CH
            pltpu.sync_copy(idx_hbm.at[pl.ds(base, CH)], idx_v)
            pltpu.async_copy(table_hbm.at[idx_v], rows_v, sem).wait()
            pltpu.sync_copy(rows_v, out_hbm.at[pl.ds(base, CH)])
            return carry

        lax.fori_loop(0, NCH, body, 0)

    @functools.partial(
        pl.kernel,
        mesh=mesh,
        out_type=jax.ShapeDtypeStruct((NC, NPAD, D), jnp.float32),
        scratch_types=[
            pltpu.VMEM((CH,), jnp.int32),
            pltpu.VMEM((CH, D), jnp.float32),
            pltpu.VMEM_SHARED((NPAD, D), jnp.float32),
            pltpu.SemaphoreType.DMA,
        ],
    )
    def sc_scatter_add(m_hbm, dst_hbm, zeros_hbm, out_hbm, idx_v, rows_v, accum, sem):
        c = lax.axis_index("c")
        s = lax.axis_index("s")
        wid = s * NC + c

        pltpu.sync_copy(zeros_hbm.at[pl.ds(s * RPS, RPS)], accum.at[pl.ds(s * RPS, RPS)])
        plsc.subcore_barrier()

        def body(i, carry):
            base = wid * EPW + i * CH
            pltpu.sync_copy(dst_hbm.at[pl.ds(base, CH)], idx_v)
            pltpu.sync_copy(m_hbm.at[pl.ds(base, CH)], rows_v)
            pltpu.sync_copy(rows_v, accum.at[idx_v], add=True)
            return carry

        lax.fori_loop(0, NCH, body, 0)
        plsc.subcore_barrier()

        pltpu.sync_copy(accum.at[pl.ds(s * RPS, RPS)], out_hbm.at[c, pl.ds(s * RPS, RPS)])

    return sc_gather, sc_scatter_add


def kernel(node_x, edge_index, edge_attr, params):
    node_x = jnp.nan_to_num(node_x.astype(jnp.float32), nan=0.0, posinf=0.0, neginf=0.0)
    edge_attr = jnp.nan_to_num(edge_attr.astype(jnp.float32), nan=0.0, posinf=0.0, neginf=0.0)
    src = edge_index[0].astype(jnp.int32)
    dst = edge_index[1].astype(jnp.int32)

    p = params
    nm, mm, um = p['node_mlp'], p['msg_mlp'], p['up_mlp']
    lng = p['ln_g'].reshape(1, D)
    lnb = p['ln_b'].reshape(1, D)

    def b(v):
        return v.reshape(1, D)

    wm = mm['W1'][:D]
    w1e = mm['W1'][D:]
    w1h = um['W1'][:D]
    w1a = um['W1'][D:]

    H, A = _node_init_call(node_x, nm['W1'], b(nm['b1']), nm['W2'], b(nm['b2']),
                           nm['W3'], b(nm['b3']), lng, lnb, wm, b(mm['b1']))

    sc_gather, sc_scatter_add = _sc_calls()
    zeros = jnp.zeros((NPAD, D), jnp.float32)
    gsum = None
    for _ in range(N_LAYERS):
        G = sc_gather(A, src)
        M = _edge_call(G, edge_attr, w1e, mm['W2'], b(mm['b2']), mm['W3'], b(mm['b3']))
        parts = sc_scatter_add(M, dst, zeros)
        H, A, gsum = _update_call(H, parts[0, :N], parts[1, :N],
                                  w1h, w1a, b(um['b1']), um['W2'], b(um['b2']),
                                  um['W3'], b(um['b3']), lng, lnb, wm, b(mm['b1']))

    g = gsum[0] / jnp.float32(N)
    return (H, g)


# speedup vs baseline: 3.5447x; 1.5504x over previous
import functools

import jax
import jax.numpy as jnp
from jax import lax
from jax.experimental import pallas as pl
from jax.experimental.pallas import tpu as pltpu
from jax.experimental.pallas import tpu_sc as plsc

N = 10000
E = 320000
D = 128
ED = 16
N_LAYERS = 3
EPS = 1e-5

NB = 1000
EB = 4000

NC = 2
NS = 16
NW = NC * NS
EPW = E // NW
CH = 40
NCH = EPW // CH
NBUF = 5
NG = NCH // NBUF
SNB = 2
SNG = NCH // SNB
NPAD = 10240
RPS = NPAD // NS


def _ln(h, g, b):
    mu = jnp.mean(h, axis=1, keepdims=True)
    var = jnp.mean((h - mu) ** 2, axis=1, keepdims=True)
    return (h - mu) * lax.rsqrt(var + EPS) * g + b


def _node_init_body(x_ref, w1, b1, w2, b2, w3, b3, lng, lnb, wm, bm,
                    h_out, a_out):
    x = x_ref[...]
    h = jnp.maximum(jnp.dot(x, w1[...], preferred_element_type=jnp.float32) + b1[...], 0.0)
    h = jnp.maximum(jnp.dot(h, w2[...], preferred_element_type=jnp.float32) + b2[...], 0.0)
    h = jnp.dot(h, w3[...], preferred_element_type=jnp.float32) + b3[...]
    hn = _ln(h, lng[...], lnb[...])
    h_out[...] = hn
    a_out[...] = jnp.dot(hn, wm[...], preferred_element_type=jnp.float32) + bm[...]


def _edge_body(g_ref, ea_ref, w1e, w2, b2, w3, b3, m_out):
    h1 = jnp.maximum(
        g_ref[...] + jnp.dot(ea_ref[...], w1e[...], preferred_element_type=jnp.float32),
        0.0)
    h2 = jnp.maximum(jnp.dot(h1, w2[...], preferred_element_type=jnp.float32) + b2[...], 0.0)
    m_out[...] = jnp.dot(h2, w3[...], preferred_element_type=jnp.float32) + b3[...]


def _update_body(h_ref, p0_ref, p1_ref, w1h, w1a, b1, w2, b2, w3, b3,
                 lng, lnb, wm, bm, h_out, a_out, gsum_out):
    h = h_ref[...]
    agg = p0_ref[...] + p1_ref[...]
    u = jnp.maximum(
        jnp.dot(h, w1h[...], preferred_element_type=jnp.float32)
        + jnp.dot(agg, w1a[...], preferred_element_type=jnp.float32) + b1[...], 0.0)
    u = jnp.maximum(jnp.dot(u, w2[...], preferred_element_type=jnp.float32) + b2[...], 0.0)
    u = jnp.dot(u, w3[...], preferred_element_type=jnp.float32) + b3[...]
    hn = _ln(h + u, lng[...], lnb[...])
    h_out[...] = hn
    a_out[...] = jnp.dot(hn, wm[...], preferred_element_type=jnp.float32) + bm[...]

    @pl.when(pl.program_id(0) == 0)
    def _():
        gsum_out[...] = jnp.zeros_like(gsum_out)
    gsum_out[...] += jnp.sum(hn, axis=0, keepdims=True)


def _full(shape):
    return pl.BlockSpec(shape, lambda i: (0, 0))


def _node_init_call(x, w1, b1, w2, b2, w3, b3, lng, lnb, wm, bm):
    grid = (N // NB,)
    row = pl.BlockSpec((NB, D), lambda i: (i, 0))
    return pl.pallas_call(
        _node_init_body,
        grid=grid,
        in_specs=[row, _full((D, D)), _full((1, D)), _full((D, D)), _full((1, D)),
                  _full((D, D)), _full((1, D)), _full((1, D)), _full((1, D)),
                  _full((D, D)), _full((1, D))],
        out_specs=[row, row],
        out_shape=[jax.ShapeDtypeStruct((N, D), jnp.float32),
                   jax.ShapeDtypeStruct((N, D), jnp.float32)],
    )(x, w1, b1, w2, b2, w3, b3, lng, lnb, wm, bm)


def _edge_call(g, ea, w1e, w2, b2, w3, b3):
    grid = (E // EB,)
    row = pl.BlockSpec((EB, D), lambda i: (i, 0))
    erow = pl.BlockSpec((EB, ED), lambda i: (i, 0))
    return pl.pallas_call(
        _edge_body,
        grid=grid,
        in_specs=[row, erow, _full((ED, D)), _full((D, D)), _full((1, D)),
                  _full((D, D)), _full((1, D))],
        out_specs=row,
        out_shape=jax.ShapeDtypeStruct((E, D), jnp.float32),
    )(g, ea, w1e, w2, b2, w3, b3)


def _update_call(h, p0, p1, w1h, w1a, b1, w2, b2, w3, b3, lng, lnb, wm, bm):
    grid = (N // NB,)
    row = pl.BlockSpec((NB, D), lambda i: (i, 0))
    return pl.pallas_call(
        _update_body,
        grid=grid,
        in_specs=[row, row, row,
                  _full((D, D)), _full((D, D)), _full((1, D)),
                  _full((D, D)), _full((1, D)), _full((D, D)), _full((1, D)),
                  _full((1, D)), _full((1, D)), _full((D, D)), _full((1, D))],
        out_specs=[row, row, _full((1, D))],
        out_shape=[jax.ShapeDtypeStruct((N, D), jnp.float32),
                   jax.ShapeDtypeStruct((N, D), jnp.float32),
                   jax.ShapeDtypeStruct((1, D), jnp.float32)],
        compiler_params=pltpu.CompilerParams(
            dimension_semantics=("arbitrary",)),
    )(h, p0, p1, w1h, w1a, b1, w2, b2, w3, b3, lng, lnb, wm, bm)


@functools.cache
def _sc_calls():
    mesh = plsc.VectorSubcoreMesh(core_axis_name="c", subcore_axis_name="s")

    @functools.partial(
        pl.kernel,
        mesh=mesh,
        out_type=jax.ShapeDtypeStruct((E, D), jnp.float32),
        scratch_types=[
            pltpu.VMEM((NCH, CH), jnp.int32),
            pltpu.VMEM((2, NBUF, CH, D), jnp.float32),
            pltpu.SemaphoreType.DMA,
            pltpu.SemaphoreType.DMA,
        ],
    )
    def sc_gather(table_hbm, idx3_hbm, out_hbm, idx_v, rows_v, gsem, wsem):
        s = lax.axis_index("s")
        wid = s * NC + lax.axis_index("c")

        pltpu.sync_copy(idx3_hbm.at[wid], idx_v)

        def phase(bank, g):
            @pl.when(g >= 2)
            def _():
                for b in range(NBUF):
                    pltpu.make_async_copy(
                        rows_v.at[bank, b], out_hbm.at[pl.ds(0, CH)], wsem).wait()
            for b in range(NBUF):
                pltpu.async_copy(
                    table_hbm.at[idx_v.at[g * NBUF + b]], rows_v.at[bank, b], gsem)
            for b in range(NBUF):
                pltpu.make_async_copy(
                    table_hbm.at[pl.ds(0, CH)], rows_v.at[bank, b], gsem).wait()
            for b in range(NBUF):
                base = wid * EPW + (g * NBUF + b) * CH
                pltpu.async_copy(rows_v.at[bank, b], out_hbm.at[pl.ds(base, CH)], wsem)

        def body(t, carry):
            phase(0, 2 * t)
            phase(1, 2 * t + 1)
            return carry

        lax.fori_loop(0, NG // 2, body, 0)
        for bank in range(2):
            for b in range(NBUF):
                pltpu.make_async_copy(
                    rows_v.at[bank, b], out_hbm.at[pl.ds(0, CH)], wsem).wait()

    @functools.partial(
        pl.kernel,
        mesh=mesh,
        out_type=jax.ShapeDtypeStruct((NC, NPAD, D), jnp.float32),
        scratch_types=[
            pltpu.VMEM((2, SNB, CH), jnp.int32),
            pltpu.VMEM((2, SNB, CH, D), jnp.float32),
            pltpu.VMEM_SHARED((NPAD, D), jnp.float32),
            pltpu.SemaphoreType.DMA,
            pltpu.SemaphoreType.DMA,
        ],
    )
    def sc_scatter_add(m_hbm, dst3_hbm, zeros_hbm, out_hbm, idx_b, rows_v, accum,
                       lsem, isem):
        c = lax.axis_index("c")
        s = lax.axis_index("s")
        wid = s * NC + c
        pltpu.sync_copy(zeros_hbm.at[pl.ds(s * RPS, RPS)], accum.at[pl.ds(s * RPS, RPS)])
        plsc.subcore_barrier()

        def fire_loads(bank, g):
            for b in range(SNB):
                j = g * SNB + b
                base = wid * EPW + j * CH
                pltpu.async_copy(dst3_hbm.at[wid, j], idx_b.at[bank, b], isem)
                pltpu.async_copy(m_hbm.at[pl.ds(base, CH)], rows_v.at[bank, b], lsem)

        def drain_loads(bank):
            for b in range(SNB):
                pltpu.make_async_copy(
                    dst3_hbm.at[wid, 0], idx_b.at[bank, b], isem).wait()
                pltpu.make_async_copy(
                    m_hbm.at[pl.ds(0, CH)], rows_v.at[bank, b], lsem).wait()

        def scatter(bank, g):
            for b in range(SNB):
                pltpu.sync_copy(
                    rows_v.at[bank, b], accum.at[idx_b.at[bank, b]], add=True)

        fire_loads(0, 0)
        fire_loads(1, 1)

        def body(t, carry):
            drain_loads(0)
            scatter(0, 2 * t)
            fire_loads(0, 2 * t + 2)
            drain_loads(1)
            scatter(1, 2 * t + 1)
            fire_loads(1, 2 * t + 3)
            return carry

        T = (SNG - 3) // 2
        lax.fori_loop(0, T, body, 0)
        drain_loads(0)
        scatter(0, 2 * T)
        fire_loads(0, SNG - 1)
        drain_loads(1)
        scatter(1, 2 * T + 1)
        drain_loads(0)
        scatter(0, SNG - 1)
        plsc.subcore_barrier()
        pltpu.sync_copy(accum.at[pl.ds(s * RPS, RPS)], out_hbm.at[c, pl.ds(s * RPS, RPS)])

    return sc_gather, sc_scatter_add


def kernel(node_x, edge_index, edge_attr, params):
    node_x = jnp.nan_to_num(node_x.astype(jnp.float32), nan=0.0, posinf=0.0, neginf=0.0)
    edge_attr = jnp.nan_to_num(edge_attr.astype(jnp.float32), nan=0.0, posinf=0.0, neginf=0.0)
    src3 = edge_index[0].astype(jnp.int32).reshape(NW, NCH, CH)
    dst3 = edge_index[1].astype(jnp.int32).reshape(NW, NCH, CH)

    p = params
    nm, mm, um = p['node_mlp'], p['msg_mlp'], p['up_mlp']
    lng = p['ln_g'].reshape(1, D)
    lnb = p['ln_b'].reshape(1, D)

    def b(v):
        return v.reshape(1, D)

    wm = mm['W1'][:D]
    w1e = mm['W1'][D:]
    w1h = um['W1'][:D]
    w1a = um['W1'][D:]

    H, A = _node_init_call(node_x, nm['W1'], b(nm['b1']), nm['W2'], b(nm['b2']),
                           nm['W3'], b(nm['b3']), lng, lnb, wm, b(mm['b1']))

    sc_gather, sc_scatter_add = _sc_calls()
    zeros = jnp.zeros((NPAD, D), jnp.float32)
    gsum = None
    for _ in range(N_LAYERS):
        G = sc_gather(A, src3)
        M = _edge_call(G, edge_attr, w1e, mm['W2'], b(mm['b2']), mm['W3'], b(mm['b3']))
        parts = sc_scatter_add(M, dst3, zeros)
        H, A, gsum = _update_call(H, parts[0, :N], parts[1, :N],
                                  w1h, w1a, b(um['b1']), um['W2'], b(um['b2']),
                                  um['W3'], b(um['b3']), lng, lnb, wm, b(mm['b1']))

    g = gsum[0] / jnp.float32(N)
    return (H, g)
